# Initial kernel scaffold; baseline (speedup 1.0000x reference)
import jax
import jax.numpy as jnp
from jax.experimental import pallas as pl


def kernel(x, edge_index, Wl1, bl1, Wr1, Wl2, bl2, Wr2, Wl3, bl3, Wr3):
    raise NotImplementedError("write your pallas kernel here")


# speedup vs baseline: 3.0078x; 3.0078x over previous
import functools

import jax
import jax.numpy as jnp
from jax import lax
from jax.experimental import pallas as pl
from jax.experimental.pallas import tpu as pltpu
from jax.experimental.pallas import tpu_sc as plsc

N = 10000
NPAD = 10240
E = 160000
D = 256
HW = 128
NS = 16
EPT = E // NS
CH = 64
NFULL = EPT // CH
REM = EPT - NFULL * CH
RPT = NPAD // NS
CW = 16
MB = 512
NW = 2 * NS
CCH = 79
EPW = CCH * CH
EPAD = NW * EPW


def _make_sc_agg():
    mesh = plsc.VectorSubcoreMesh(core_axis_name="c", subcore_axis_name="s",
                                  num_cores=2, num_subcores=NS)
    out_type = jax.ShapeDtypeStruct((2, NPAD, HW), jnp.float32)
    scratch = [
        pltpu.VMEM_SHARED((NPAD, HW), jnp.float32),
        pltpu.VMEM((CH,), jnp.int32),
        pltpu.VMEM((CH,), jnp.int32),
        pltpu.VMEM((CH, HW), jnp.float32),
        pltpu.VMEM((REM,), jnp.int32),
        pltpu.VMEM((REM,), jnp.int32),
        pltpu.VMEM((REM, HW), jnp.float32),
        pltpu.SemaphoreType.DMA,
    ]

    def body(tlo, thi, src_h, dst_h, zrows_h, out,
             acc_sh, sidx, didx, rows, sidx_r, didx_r, rows_r, sem):
        c = lax.axis_index("c")
        s = lax.axis_index("s")

        pltpu.sync_copy(zrows_h, rows)
        for b in range(RPT // CH):
            off = pl.multiple_of(s * RPT + b * CH, CH)
            pltpu.sync_copy(rows, acc_sh.at[pl.ds(off, CH)])
        plsc.subcore_barrier()

        def chunk(i, carry):
            base = pl.multiple_of(s * EPT + i * CH, 8)
            pltpu.sync_copy(src_h.at[pl.ds(base, CH)], sidx)
            pltpu.sync_copy(dst_h.at[pl.ds(base, CH)], didx)

            @pl.when(c == 0)
            def _():
                pltpu.async_copy(tlo.at[sidx], rows, sem).wait()

            @pl.when(c == 1)
            def _():
                pltpu.async_copy(thi.at[sidx], rows, sem).wait()

            pltpu.sync_copy(rows, acc_sh.at[didx], add=True)
            return carry

        lax.fori_loop(0, NFULL, chunk, 0)

        if REM:
            baser = pl.multiple_of(s * EPT + NFULL * CH, 8)
            pltpu.sync_copy(src_h.at[pl.ds(baser, REM)], sidx_r)
            pltpu.sync_copy(dst_h.at[pl.ds(baser, REM)], didx_r)

            @pl.when(c == 0)
            def _():
                pltpu.async_copy(tlo.at[sidx_r], rows_r, sem).wait()

            @pl.when(c == 1)
            def _():
                pltpu.async_copy(thi.at[sidx_r], rows_r, sem).wait()

            pltpu.sync_copy(rows_r, acc_sh.at[didx_r], add=True)

        plsc.subcore_barrier()
        off = pl.multiple_of(s * RPT, 8)

        @pl.when(c == 0)
        def _():
            pltpu.sync_copy(acc_sh.at[pl.ds(off, RPT)],
                            out.at[0, pl.ds(off, RPT)])

        @pl.when(c == 1)
        def _():
            pltpu.sync_copy(acc_sh.at[pl.ds(off, RPT)],
                            out.at[1, pl.ds(off, RPT)])

    return pl.kernel(body, out_type=out_type, mesh=mesh,
                     scratch_types=scratch)


def _make_sc_counts():
    mesh = plsc.VectorSubcoreMesh(core_axis_name="c", subcore_axis_name="s",
                                  num_cores=2, num_subcores=NS)
    out_type = jax.ShapeDtypeStruct((2, NPAD, HW), jnp.float32)
    scratch = [
        pltpu.VMEM_SHARED((NPAD, HW), jnp.float32),
        pltpu.VMEM((CCH, CH), jnp.int32),
        pltpu.VMEM((CH, HW), jnp.float32),
        pltpu.VMEM((CH, HW), jnp.float32),
    ]

    def body(dstc, zrows_h, orows_h, out, cnt_sh, didx2, ones, zeros):
        c = lax.axis_index("c")
        s = lax.axis_index("s")
        pltpu.sync_copy(zrows_h, zeros)
        pltpu.sync_copy(orows_h, ones)

        @pl.when(c == 0)
        def _():
            pltpu.sync_copy(dstc.at[s], didx2)

        @pl.when(c == 1)
        def _():
            pltpu.sync_copy(dstc.at[NS + s], didx2)

        for b in range(RPT // CH):
            off = pl.multiple_of(s * RPT + b * CH, CH)
            pltpu.sync_copy(zeros, cnt_sh.at[pl.ds(off, CH)])
        plsc.subcore_barrier()

        def chunk(j, carry):
            pltpu.sync_copy(ones, cnt_sh.at[didx2.at[j]], add=True)
            return carry

        lax.fori_loop(0, CCH, chunk, 0)

        plsc.subcore_barrier()
        off = pl.multiple_of(s * RPT, 8)

        @pl.when(c == 0)
        def _():
            pltpu.sync_copy(cnt_sh.at[pl.ds(off, RPT)],
                            out.at[0, pl.ds(off, RPT)])

        @pl.when(c == 1)
        def _():
            pltpu.sync_copy(cnt_sh.at[pl.ds(off, RPT)],
                            out.at[1, pl.ds(off, RPT)])

    return pl.kernel(body, out_type=out_type, mesh=mesh,
                     scratch_types=scratch)


_make_sc_agg = functools.lru_cache(maxsize=None)(_make_sc_agg)
_make_sc_counts = functools.lru_cache(maxsize=None)(_make_sc_counts)


def _make_tc_layer(split_out):
    def body(alo, ahi, cnt, tlo, thi, Wl, bl, Wr, *outs):
        r = 1.0 / jnp.maximum(cnt[:, 0:1], 1.0)
        agg = jnp.concatenate([alo[...], ahi[...]], axis=1) * r
        h = jnp.concatenate([tlo[...], thi[...]], axis=1)
        o = (lax.dot_general(agg, Wl[...], (((1,), (1,)), ((), ())),
                             preferred_element_type=jnp.float32)
             + bl[...]
             + lax.dot_general(h, Wr[...], (((1,), (1,)), ((), ())),
                               preferred_element_type=jnp.float32))
        o = jnp.maximum(o, 0.0)
        if split_out:
            outs[0][...] = o[:, :HW]
            outs[1][...] = o[:, HW:]
        else:
            outs[0][...] = o

    grid = (NPAD // MB,)
    in_specs = [
        pl.BlockSpec((MB, HW), lambda i: (i, 0)),
        pl.BlockSpec((MB, HW), lambda i: (i, 0)),
        pl.BlockSpec((MB, CW), lambda i: (i, 0)),
        pl.BlockSpec((MB, HW), lambda i: (i, 0)),
        pl.BlockSpec((MB, HW), lambda i: (i, 0)),
        pl.BlockSpec((D, D), lambda i: (0, 0)),
        pl.BlockSpec((1, D), lambda i: (0, 0)),
        pl.BlockSpec((D, D), lambda i: (0, 0)),
    ]
    if split_out:
        out_specs = [pl.BlockSpec((MB, HW), lambda i: (i, 0))] * 2
        out_shape = [jax.ShapeDtypeStruct((NPAD, HW), jnp.float32)] * 2
    else:
        out_specs = pl.BlockSpec((MB, D), lambda i: (i, 0))
        out_shape = jax.ShapeDtypeStruct((NPAD, D), jnp.float32)
    return pl.pallas_call(body, grid=grid, in_specs=in_specs,
                          out_specs=out_specs, out_shape=out_shape)


_tc_split = _make_tc_layer(True)
_tc_full = _make_tc_layer(False)


@jax.jit
def kernel(x, edge_index, Wl1, bl1, Wr1, Wl2, bl2, Wr2, Wl3, bl3, Wr3):
    ei = edge_index.astype(jnp.int32)
    src = ei[0]
    dst = ei[1]
    pad = jnp.zeros((NPAD - N, HW), jnp.float32)
    t0lo = jnp.concatenate([x[:, :HW], pad], axis=0)
    t0hi = jnp.concatenate([x[:, HW:], pad], axis=0)
    zrows = jnp.zeros((CH, HW), jnp.float32)
    orows = jnp.ones((CH, HW), jnp.float32)
    dstc = jnp.concatenate(
        [dst, jnp.full((EPAD - E,), NPAD - 1, jnp.int32)]).reshape(
            NW, CCH, CH)

    cnt2 = _make_sc_counts()(dstc, zrows, orows)
    cnt1d = cnt2[0, :, 0] + cnt2[1, :, 0]
    cnt16 = jnp.broadcast_to(cnt1d[:, None], (NPAD, CW))

    sum1 = _make_sc_agg()(t0lo, t0hi, src, dst, zrows)
    t1lo, t1hi = _tc_split(sum1[0], sum1[1], cnt16,
                           t0lo, t0hi, Wl1, bl1.reshape(1, D), Wr1)
    sum2 = _make_sc_agg()(t1lo, t1hi, src, dst, zrows)
    t2lo, t2hi = _tc_split(sum2[0], sum2[1], cnt16,
                           t1lo, t1hi, Wl2, bl2.reshape(1, D), Wr2)
    sum3 = _make_sc_agg()(t2lo, t2hi, src, dst, zrows)
    out = _tc_full(sum3[0], sum3[1], cnt16,
                   t2lo, t2hi, Wl3, bl3.reshape(1, D), Wr3)
    return out[:N]


# speedup vs baseline: 4.2322x; 1.4071x over previous
import functools

import jax
import jax.numpy as jnp
from jax import lax
from jax.experimental import pallas as pl
from jax.experimental.pallas import tpu as pltpu
from jax.experimental.pallas import tpu_sc as plsc

N = 10000
NPAD = 10240
E = 160000
D = 256
HW = 128
NS = 16
CH = 64
NC2 = 158
RPT = NPAD // NS
CW = 16
MB = 512
NW = 2 * NS
CCH = 79
EPAD = NS * NC2 * CH


def _make_sc_agg():
    mesh = plsc.VectorSubcoreMesh(core_axis_name="c", subcore_axis_name="s",
                                  num_cores=2, num_subcores=NS)
    out_type = jax.ShapeDtypeStruct((2, NPAD, HW), jnp.float32)
    scratch = [
        pltpu.VMEM_SHARED((NPAD, HW), jnp.float32),
        [pltpu.VMEM((CH,), jnp.int32)] * 2,
        [pltpu.VMEM((CH,), jnp.int32)] * 2,
        [pltpu.VMEM((CH, HW), jnp.float32)] * 2,
        [pltpu.SemaphoreType.DMA] * 6,
    ]

    def body(tlo, thi, srcP, dstP, zrows_h, out,
             acc_sh, sidx, didx, rows, sems):
        c = lax.axis_index("c")
        s = lax.axis_index("s")
        isem = sems[0:2]
        dsem = sems[2:4]
        gsem = sems[4:6]

        pltpu.sync_copy(zrows_h, rows[0])
        for b in range(RPT // CH):
            off = pl.multiple_of(s * RPT + b * CH, CH)
            pltpu.sync_copy(rows[0], acc_sh.at[pl.ds(off, CH)])
        plsc.subcore_barrier()

        def start_idx(j, p):
            base = pl.multiple_of((s * NC2 + j) * CH, 8)
            pltpu.async_copy(srcP.at[pl.ds(base, CH)], sidx[p], isem[p])
            pltpu.async_copy(dstP.at[pl.ds(base, CH)], didx[p], dsem[p])

        def start_gather(p):
            pltpu.make_async_copy(srcP.at[pl.ds(0, CH)], sidx[p],
                                  isem[p]).wait()

            @pl.when(c == 0)
            def _():
                pltpu.async_copy(tlo.at[sidx[p]], rows[p], gsem[p])

            @pl.when(c == 1)
            def _():
                pltpu.async_copy(thi.at[sidx[p]], rows[p], gsem[p])

        start_idx(0, 0)
        start_idx(1, 1)
        start_gather(0)
        start_gather(1)

        def chunk(j2, carry):
            for p in range(2):
                j = 2 * j2 + p
                pltpu.make_async_copy(tlo.at[sidx[p]], rows[p],
                                      gsem[p]).wait()
                pltpu.make_async_copy(dstP.at[pl.ds(0, CH)], didx[p],
                                      dsem[p]).wait()
                pltpu.sync_copy(rows[p], acc_sh.at[didx[p]], add=True)

                @pl.when(j + 2 < NC2)
                def _(j=j, p=p):
                    start_idx(j + 2, p)
                    start_gather(p)
            return carry

        lax.fori_loop(0, NC2 // 2, chunk, 0)

        plsc.subcore_barrier()
        off = pl.multiple_of(s * RPT, 8)

        @pl.when(c == 0)
        def _():
            pltpu.sync_copy(acc_sh.at[pl.ds(off, RPT)],
                            out.at[0, pl.ds(off, RPT)])

        @pl.when(c == 1)
        def _():
            pltpu.sync_copy(acc_sh.at[pl.ds(off, RPT)],
                            out.at[1, pl.ds(off, RPT)])

    return pl.kernel(body, out_type=out_type, mesh=mesh,
                     scratch_types=scratch)


def _make_sc_counts():
    mesh = plsc.VectorSubcoreMesh(core_axis_name="c", subcore_axis_name="s",
                                  num_cores=2, num_subcores=NS)
    out_type = jax.ShapeDtypeStruct((2, NPAD, HW), jnp.float32)
    scratch = [
        pltpu.VMEM_SHARED((NPAD, HW), jnp.float32),
        pltpu.VMEM((CCH, CH), jnp.int32),
        pltpu.VMEM((CH, HW), jnp.float32),
        pltpu.VMEM((CH, HW), jnp.float32),
    ]

    def body(dstc, zrows_h, orows_h, out, cnt_sh, didx2, ones, zeros):
        c = lax.axis_index("c")
        s = lax.axis_index("s")
        pltpu.sync_copy(zrows_h, zeros)
        pltpu.sync_copy(orows_h, ones)

        @pl.when(c == 0)
        def _():
            pltpu.sync_copy(dstc.at[s], didx2)

        @pl.when(c == 1)
        def _():
            pltpu.sync_copy(dstc.at[NS + s], didx2)

        for b in range(RPT // CH):
            off = pl.multiple_of(s * RPT + b * CH, CH)
            pltpu.sync_copy(zeros, cnt_sh.at[pl.ds(off, CH)])
        plsc.subcore_barrier()

        def chunk(j, carry):
            pltpu.sync_copy(ones, cnt_sh.at[didx2.at[j]], add=True)
            return carry

        lax.fori_loop(0, CCH, chunk, 0)

        plsc.subcore_barrier()
        off = pl.multiple_of(s * RPT, 8)

        @pl.when(c == 0)
        def _():
            pltpu.sync_copy(cnt_sh.at[pl.ds(off, RPT)],
                            out.at[0, pl.ds(off, RPT)])

        @pl.when(c == 1)
        def _():
            pltpu.sync_copy(cnt_sh.at[pl.ds(off, RPT)],
                            out.at[1, pl.ds(off, RPT)])

    return pl.kernel(body, out_type=out_type, mesh=mesh,
                     scratch_types=scratch)


_make_sc_agg = functools.lru_cache(maxsize=None)(_make_sc_agg)
_make_sc_counts = functools.lru_cache(maxsize=None)(_make_sc_counts)


def _make_tc_layer(split_out):
    def body(alo, ahi, cnt, tlo, thi, Wl, bl, Wr, *outs):
        r = 1.0 / jnp.maximum(cnt[:, 0:1], 1.0)
        agg = jnp.concatenate([alo[...], ahi[...]], axis=1) * r
        h = jnp.concatenate([tlo[...], thi[...]], axis=1)
        o = (lax.dot_general(agg, Wl[...], (((1,), (1,)), ((), ())),
                             preferred_element_type=jnp.float32)
             + bl[...]
             + lax.dot_general(h, Wr[...], (((1,), (1,)), ((), ())),
                               preferred_element_type=jnp.float32))
        o = jnp.maximum(o, 0.0)
        if split_out:
            outs[0][...] = o[:, :HW]
            outs[1][...] = o[:, HW:]
        else:
            outs[0][...] = o

    grid = (NPAD // MB,)
    in_specs = [
        pl.BlockSpec((MB, HW), lambda i: (i, 0)),
        pl.BlockSpec((MB, HW), lambda i: (i, 0)),
        pl.BlockSpec((MB, CW), lambda i: (i, 0)),
        pl.BlockSpec((MB, HW), lambda i: (i, 0)),
        pl.BlockSpec((MB, HW), lambda i: (i, 0)),
        pl.BlockSpec((D, D), lambda i: (0, 0)),
        pl.BlockSpec((1, D), lambda i: (0, 0)),
        pl.BlockSpec((D, D), lambda i: (0, 0)),
    ]
    if split_out:
        out_specs = [pl.BlockSpec((MB, HW), lambda i: (i, 0))] * 2
        out_shape = [jax.ShapeDtypeStruct((NPAD, HW), jnp.float32)] * 2
    else:
        out_specs = pl.BlockSpec((MB, D), lambda i: (i, 0))
        out_shape = jax.ShapeDtypeStruct((NPAD, D), jnp.float32)
    return pl.pallas_call(body, grid=grid, in_specs=in_specs,
                          out_specs=out_specs, out_shape=out_shape)


_tc_split = _make_tc_layer(True)
_tc_full = _make_tc_layer(False)


@jax.jit
def kernel(x, edge_index, Wl1, bl1, Wr1, Wl2, bl2, Wr2, Wl3, bl3, Wr3):
    ei = edge_index.astype(jnp.int32)
    src = ei[0]
    dst = ei[1]
    pad = jnp.zeros((NPAD - N, HW), jnp.float32)
    t0lo = jnp.concatenate([x[:, :HW], pad], axis=0)
    t0hi = jnp.concatenate([x[:, HW:], pad], axis=0)
    zrows = jnp.zeros((CH, HW), jnp.float32)
    orows = jnp.ones((CH, HW), jnp.float32)
    srcP = jnp.concatenate([src, jnp.zeros((EPAD - E,), jnp.int32)])
    dstP = jnp.concatenate(
        [dst, jnp.full((EPAD - E,), NPAD - 1, jnp.int32)])
    dstc = dstP.reshape(NW, CCH, CH)

    cnt2 = _make_sc_counts()(dstc, zrows, orows)
    cnt1d = cnt2[0, :, 0] + cnt2[1, :, 0]
    cnt16 = jnp.broadcast_to(cnt1d[:, None], (NPAD, CW))

    sum1 = _make_sc_agg()(t0lo, t0hi, srcP, dstP, zrows)
    t1lo, t1hi = _tc_split(sum1[0], sum1[1], cnt16,
                           t0lo, t0hi, Wl1, bl1.reshape(1, D), Wr1)
    sum2 = _make_sc_agg()(t1lo, t1hi, srcP, dstP, zrows)
    t2lo, t2hi = _tc_split(sum2[0], sum2[1], cnt16,
                           t1lo, t1hi, Wl2, bl2.reshape(1, D), Wr2)
    sum3 = _make_sc_agg()(t2lo, t2hi, srcP, dstP, zrows)
    out = _tc_full(sum3[0], sum3[1], cnt16,
                   t2lo, t2hi, Wl3, bl3.reshape(1, D), Wr3)
    return out[:N]
